# baseline (speedup 1.0000x reference)
import functools
import jax
import jax.numpy as jnp
from jax import lax
from jax.experimental import pallas as pl
from jax.experimental.pallas import tpu as pltpu
from jax.experimental.pallas import tpu_sc as plsc

NC = 2
NS = 16
NW = NC * NS
CHUNK = 80
NBUF = 4
NISLOT = 8
UNROLL = 8


def _matmul_call(x, W, b2):
    n, d = x.shape
    bm = 2000

    def body(x_ref, w_ref, b_ref, o_ref):
        o_ref[...] = lax.dot_general(
            x_ref[...], w_ref[...], (((1,), (1,)), ((), ())),
            preferred_element_type=jnp.float32) + b_ref[...]

    return pl.pallas_call(
        body,
        grid=(n // bm,),
        in_specs=[
            pl.BlockSpec((bm, d), lambda i: (i, 0)),
            pl.BlockSpec((d, d), lambda i: (0, 0)),
            pl.BlockSpec((1, d), lambda i: (0, 0)),
        ],
        out_specs=pl.BlockSpec((bm, d), lambda i: (i, 0)),
        out_shape=jax.ShapeDtypeStruct((n, d), jnp.float32),
    )(x, W, b2)


def _combine_call(partial, n):
    _, _, d = partial.shape
    bm = 2000

    def body(p_ref, o_ref):
        o_ref[...] = p_ref[0] + p_ref[1]

    return pl.pallas_call(
        body,
        grid=(n // bm,),
        in_specs=[pl.BlockSpec((2, bm, d), lambda i: (0, i, 0))],
        out_specs=pl.BlockSpec((bm, d), lambda i: (i, 0)),
        out_shape=jax.ShapeDtypeStruct((n, d), jnp.float32),
    )(partial)


def _sc_scatter_call(h_a, eidx, n_rows_sh, nch):
    n, d = h_a.shape
    rpt = n_rows_sh // NS
    nzf = rpt // CHUNK
    nzr = rpt - nzf * CHUNK

    mesh = plsc.VectorSubcoreMesh(core_axis_name="c", subcore_axis_name="s")

    @functools.partial(
        pl.kernel,
        out_type=jax.ShapeDtypeStruct((NC, n_rows_sh, d), jnp.float32),
        mesh=mesh,
        scratch_types=[
            *[pltpu.VMEM((CHUNK, d), jnp.float32) for _ in range(NBUF)],
            *[pltpu.VMEM((2, CHUNK), jnp.int32) for _ in range(NISLOT)],
            pltpu.VMEM_SHARED((n_rows_sh, d), jnp.float32),
            *[pltpu.SemaphoreType.DMA for _ in range(2 * NBUF + NISLOT + 1)],
        ],
    )
    def sc_kernel(ha_hbm, eidx_hbm, out_hbm, r0, r1, r2, r3,
                  i0, i1, i2, i3, i4, i5, i6, i7, hp_sh,
                  g0, g1, g2, g3, s0, s1, s2, s3,
                  q0, q1, q2, q3, q4, q5, q6, q7, zsem):
        rows = (r0, r1, r2, r3)
        idxb = (i0, i1, i2, i3, i4, i5, i6, i7)
        gsem = (g0, g1, g2, g3)
        ssem = (s0, s1, s2, s3)
        isem = (q0, q1, q2, q3, q4, q5, q6, q7)
        cid = lax.axis_index("c")
        sid = lax.axis_index("s")
        wid = cid * NS + sid
        base_r = sid * rpt

        def i_start(c, islot):
            pltpu.async_copy(eidx_hbm.at[wid].at[c], idxb[islot],
                             isem[islot])

        def i_wait(c, islot):
            pltpu.make_async_copy(eidx_hbm.at[wid].at[c], idxb[islot],
                                  isem[islot]).wait()

        def g_start(islot, rslot):
            pltpu.async_copy(
                ha_hbm.at[idxb[islot].at[0]], rows[rslot], gsem[rslot])

        def g_wait(islot, rslot):
            pltpu.make_async_copy(
                ha_hbm.at[idxb[islot].at[0]], rows[rslot],
                gsem[rslot]).wait()

        def s_start(islot, rslot):
            pltpu.async_copy(
                rows[rslot], hp_sh.at[idxb[islot].at[1]], ssem[rslot],
                add=True)

        def s_wait(islot, rslot):
            pltpu.make_async_copy(
                rows[rslot], hp_sh.at[idxb[islot].at[1]],
                ssem[rslot]).wait()

        def zrow(i, carry):
            for rb in rows:
                for j in range(d // 16):
                    rb[i, pl.ds(j * 16, 16)] = jnp.zeros((16,), jnp.float32)
            return carry
        lax.fori_loop(0, CHUNK, zrow, 0)
        zdescs = []
        for k in range(nzf):
            zdescs.append((rows[k % NBUF],
                           hp_sh.at[pl.ds(base_r + k * CHUNK, CHUNK)]))
        if nzr:
            zdescs.append((rows[nzf % NBUF].at[pl.ds(0, nzr)],
                           hp_sh.at[pl.ds(base_r + nzf * CHUNK, nzr)]))
        for src_ref, dst_ref in zdescs:
            pltpu.async_copy(src_ref, dst_ref, zsem)
        for k in range(4):
            i_start(k, k)
        for src_ref, dst_ref in zdescs:
            pltpu.make_async_copy(src_ref, dst_ref, zsem).wait()
        plsc.subcore_barrier()

        i_wait(0, 0)
        g_start(0, 0)
        i_wait(1, 1)
        g_start(1, 1)

        def outer(t, carry):
            for u in range(UNROLL):
                c = UNROLL * t + u

                @pl.when(c >= 2)
                def _():
                    s_wait((u + 6) % NISLOT, (u + 2) % NBUF)

                @pl.when(c + 4 < nch)
                def _():
                    i_start(c + 4, (u + 4) % NISLOT)

                @pl.when(c + 2 < nch)
                def _():
                    i_wait(c + 2, (u + 2) % NISLOT)
                    g_start((u + 2) % NISLOT, (u + 2) % NBUF)

                g_wait(u % NISLOT, u % NBUF)
                s_start(u % NISLOT, u % NBUF)
            return carry

        lax.fori_loop(0, nch // UNROLL, outer, 0)
        for k in (nch - 2, nch - 1):
            s_wait(k % NISLOT, k % NBUF)
        plsc.subcore_barrier()
        pltpu.sync_copy(hp_sh.at[pl.ds(base_r, rpt)],
                        out_hbm.at[cid].at[pl.ds(base_r, rpt)])

    return sc_kernel(h_a, eidx)


def kernel(x, edge_index, W, b):
    n, d = x.shape
    e = edge_index.shape[1]

    h_a = _matmul_call(x, W, b.reshape(1, d))

    rpt = -(-n // NS)
    if NS * rpt <= n:
        rpt += 1
    rpt = -(-rpt // 8) * 8
    n_rows_sh = NS * rpt

    ncb = NW * CHUNK
    nch = -(-e // ncb)
    nch = (nch + UNROLL - 1) // UNROLL * UNROLL
    e_pad = nch * ncb

    src = edge_index[0]
    dst = edge_index[1]
    if e_pad > e:
        pad = e_pad - e
        src = jnp.concatenate([src, jnp.zeros((pad,), src.dtype)])
        dst = jnp.concatenate(
            [dst, jnp.full((pad,), n_rows_sh - 1, dst.dtype)])
    eidx = jnp.stack(
        [src.reshape(NW, nch, CHUNK), dst.reshape(NW, nch, CHUNK)], axis=2)

    partial = _sc_scatter_call(h_a, eidx, n_rows_sh, nch)
    h_p = _combine_call(partial, n)
    return (h_a, h_p)


# speedup vs baseline: 1.8027x; 1.8027x over previous
import functools
import jax
import jax.numpy as jnp
from jax import lax
from jax.experimental import pallas as pl
from jax.experimental.pallas import tpu as pltpu
from jax.experimental.pallas import tpu_sc as plsc

NC = 2
NS = 16
NW = NC * NS
CHUNK = 120
NBUF = 3
NISLOT = 4
UNROLL = 12


def _matmul_call(x, W, b2):
    n, d = x.shape
    bm = n

    def body(x_ref, w_ref, b_ref, o_ref):
        o_ref[...] = lax.dot_general(
            x_ref[...], w_ref[...], (((1,), (1,)), ((), ())),
            preferred_element_type=jnp.float32) + b_ref[...]

    return pl.pallas_call(
        body,
        grid=(n // bm,),
        in_specs=[
            pl.BlockSpec((bm, d), lambda i: (i, 0)),
            pl.BlockSpec((d, d), lambda i: (0, 0)),
            pl.BlockSpec((1, d), lambda i: (0, 0)),
        ],
        out_specs=pl.BlockSpec((bm, d), lambda i: (i, 0)),
        out_shape=jax.ShapeDtypeStruct((n, d), jnp.float32),
    )(x, W, b2)


def _combine_call(partial, n):
    _, _, d = partial.shape
    bm = n

    def body(p_ref, o_ref):
        o_ref[...] = p_ref[0] + p_ref[1]

    return pl.pallas_call(
        body,
        grid=(n // bm,),
        in_specs=[pl.BlockSpec((2, bm, d), lambda i: (0, i, 0))],
        out_specs=pl.BlockSpec((bm, d), lambda i: (i, 0)),
        out_shape=jax.ShapeDtypeStruct((n, d), jnp.float32),
    )(partial)


def _sc_scatter_call(h_a, eidx, n_rows_sh, nch):
    n, d = h_a.shape
    rpt = n_rows_sh // NS
    nzf = rpt // CHUNK
    nzr = rpt - nzf * CHUNK

    mesh = plsc.VectorSubcoreMesh(core_axis_name="c", subcore_axis_name="s")

    @functools.partial(
        pl.kernel,
        out_type=jax.ShapeDtypeStruct((NC, n_rows_sh, d), jnp.float32),
        mesh=mesh,
        scratch_types=[
            *[pltpu.VMEM((CHUNK, d), jnp.float32) for _ in range(NBUF)],
            *[pltpu.VMEM((2, CHUNK), jnp.int32) for _ in range(NISLOT)],
            pltpu.VMEM_SHARED((n_rows_sh, d), jnp.float32),
            *[pltpu.SemaphoreType.DMA for _ in range(2 * NBUF + NISLOT + 1)],
        ],
    )
    def sc_kernel(ha_hbm, eidx_hbm, out_hbm, r0, r1, r2, i0, i1, i2, i3,
                  hp_sh, g0, g1, g2, s0, s1, s2, q0, q1, q2, q3, zsem):
        rows = (r0, r1, r2)
        idxb = (i0, i1, i2, i3)
        gsem = (g0, g1, g2)
        ssem = (s0, s1, s2)
        isem = (q0, q1, q2, q3)
        cid = lax.axis_index("c")
        sid = lax.axis_index("s")
        wid = cid * NS + sid
        base_r = sid * rpt

        def i_start(c, islot):
            pltpu.async_copy(eidx_hbm.at[wid].at[c], idxb[islot],
                             isem[islot])

        def i_wait(c, islot):
            pltpu.make_async_copy(eidx_hbm.at[wid].at[c], idxb[islot],
                                  isem[islot]).wait()

        def g_start(islot, rslot):
            pltpu.async_copy(
                ha_hbm.at[idxb[islot].at[0]], rows[rslot], gsem[rslot])

        def g_wait(islot, rslot):
            pltpu.make_async_copy(
                ha_hbm.at[idxb[islot].at[0]], rows[rslot],
                gsem[rslot]).wait()

        def s_start(islot, rslot):
            pltpu.async_copy(
                rows[rslot], hp_sh.at[idxb[islot].at[1]], ssem[rslot],
                add=True)

        def s_wait(islot, rslot):
            pltpu.make_async_copy(
                rows[rslot], hp_sh.at[idxb[islot].at[1]],
                ssem[rslot]).wait()

        def zrow(i, carry):
            for rb in rows:
                for j in range(d // 16):
                    rb[i, pl.ds(j * 16, 16)] = jnp.zeros((16,), jnp.float32)
            return carry
        lax.fori_loop(0, CHUNK, zrow, 0)
        zdescs = []
        for k in range(nzf):
            zdescs.append((rows[k % NBUF],
                           hp_sh.at[pl.ds(base_r + k * CHUNK, CHUNK)]))
        if nzr:
            zdescs.append((rows[nzf % NBUF].at[pl.ds(0, nzr)],
                           hp_sh.at[pl.ds(base_r + nzf * CHUNK, nzr)]))
        for src_ref, dst_ref in zdescs:
            pltpu.async_copy(src_ref, dst_ref, zsem)
        i_start(0, 0)
        i_start(1, 1)
        for src_ref, dst_ref in zdescs:
            pltpu.make_async_copy(src_ref, dst_ref, zsem).wait()
        plsc.subcore_barrier()

        i_wait(0, 0)
        g_start(0, 0)

        def outer(t, carry):
            for u in range(UNROLL):
                c = UNROLL * t + u

                @pl.when(c >= 2)
                def _():
                    s_wait((u + 2) % NISLOT, (u + 1) % NBUF)

                @pl.when(c + 2 < nch)
                def _():
                    i_start(c + 2, (u + 2) % NISLOT)

                @pl.when(c + 1 < nch)
                def _():
                    i_wait(c + 1, (u + 1) % NISLOT)
                    g_start((u + 1) % NISLOT, (u + 1) % NBUF)

                g_wait(u % NISLOT, u % NBUF)
                s_start(u % NISLOT, u % NBUF)
            return carry

        lax.fori_loop(0, nch // UNROLL, outer, 0)
        for k in (nch - 2, nch - 1):
            s_wait(k % NISLOT, k % NBUF)
        plsc.subcore_barrier()
        pltpu.sync_copy(hp_sh.at[pl.ds(base_r, rpt)],
                        out_hbm.at[cid].at[pl.ds(base_r, rpt)])

    return sc_kernel(h_a, eidx)


def kernel(x, edge_index, W, b):
    n, d = x.shape
    e = edge_index.shape[1]

    h_a = _matmul_call(x, W, b.reshape(1, d))

    rpt = -(-n // NS)
    if NS * rpt <= n:
        rpt += 1
    rpt = -(-rpt // 8) * 8
    n_rows_sh = NS * rpt

    ncb = NW * CHUNK
    nch = -(-e // ncb)
    nch = (nch + UNROLL - 1) // UNROLL * UNROLL
    e_pad = nch * ncb

    src = edge_index[0]
    dst = edge_index[1]
    if e_pad > e:
        pad = e_pad - e
        src = jnp.concatenate([src, jnp.zeros((pad,), src.dtype)])
        dst = jnp.concatenate(
            [dst, jnp.full((pad,), n_rows_sh - 1, dst.dtype)])
    eidx = jnp.stack(
        [src.reshape(NW, nch, CHUNK), dst.reshape(NW, nch, CHUNK)], axis=2)

    partial = _sc_scatter_call(h_a, eidx, n_rows_sh, nch)
    h_p = _combine_call(partial, n)
    return (h_a, h_p)
